# baseline (speedup 1.0000x reference)
import functools

import numpy as np

import jax
import jax.numpy as jnp
from jax import lax
from jax.experimental import pallas as pl
from jax.experimental.pallas import tpu as pltpu

H = W = 28
WPAD = 32
NKER = 5
HP = 14
JW = 16
GROUP = NKER * JW
MROWS = 4 * GROUP
KCOLS = 4 * WPAD
KFEAT = HP * GROUP
NOUT = 10
NOUT_PAD = 16
BLOCK_B = 512


def _conv_scatter_indices():
    ms, ks, cs, hs, ws = [], [], [], [], []
    for g in range(4):
        r, par = g // 2, g % 2
        for co in range(NKER):
            for wq in range(HP):
                m = g * GROUP + co * JW + wq
                for kh in range(3):
                    for kw in range(3):
                        wi = 2 * wq + par - 1 + kw
                        if 0 <= wi < W:
                            ms.append(m)
                            ks.append((r + kh) * WPAD + wi)
                            cs.append(co)
                            hs.append(kh)
                            ws.append(kw)
    return (np.array(ms), np.array(ks), np.array(cs), np.array(hs),
            np.array(ws))

_MIDX, _KIDX, _CIDX, _HIDX, _WIDX = _conv_scatter_indices()


def _body(x_ref,
          a_ref,
          wfc_ref,
          bias_ref,
          out_ref,
          xp_ref,
          feat_ref):
    bb = x_ref.shape[0]
    row = lax.broadcasted_iota(jnp.int32, (bb, bb), 0)
    col = lax.broadcasted_iota(jnp.int32, (bb, bb), 1)
    eye = (row == col).astype(jnp.bfloat16)

    xbf = x_ref[:, 0].astype(jnp.bfloat16)
    xt = lax.dot_general(xbf, eye, (((0,), (0,)), ((), ())),
                         preferred_element_type=jnp.float32)

    xp_ref[0] = jnp.zeros((WPAD, bb), jnp.bfloat16)
    xp_ref[H + 1] = jnp.zeros((WPAD, bb), jnp.bfloat16)
    xp_ref[1:H + 1, W:, :] = jnp.zeros((H, WPAD - W, bb), jnp.bfloat16)
    xp_ref[1:H + 1, 0:W, :] = xt.astype(jnp.bfloat16)

    for i in range(HP):
        xs = xp_ref[2 * i:2 * i + 4].reshape(KCOLS, bb)
        s = jnp.dot(a_ref[...], xs,
                    preferred_element_type=jnp.float32)
        p = jnp.maximum(jnp.maximum(s[0:GROUP], s[GROUP:2 * GROUP]),
                        jnp.maximum(s[2 * GROUP:3 * GROUP], s[3 * GROUP:]))
        p = jnp.maximum(p, 0.0)
        feat_ref[i * GROUP:(i + 1) * GROUP, :] = p.astype(jnp.bfloat16)

    out_ref[...] = (
        jnp.dot(wfc_ref[...], feat_ref[...],
                preferred_element_type=jnp.float32) + bias_ref[...])


@functools.partial(jax.jit, static_argnames=("block_b",))
def _forward(x, conv_w, fc_w, fc_b, *, block_b=BLOCK_B):
    B = x.shape[0]
    assert x.shape[1:] == (1, H, W)
    b_pad = ((B + block_b - 1) // block_b) * block_b

    xsq = x.astype(jnp.float32)
    if b_pad != B:
        xsq = jnp.pad(xsq, ((0, b_pad - B), (0, 0), (0, 0), (0, 0)))

    cw = conv_w.astype(jnp.float32)
    vals = cw[_CIDX, 0, _HIDX, _WIDX]
    amat = jnp.zeros((MROWS, KCOLS), jnp.float32).at[_MIDX, _KIDX].set(vals)
    amat = amat.astype(jnp.bfloat16)

    wfc = fc_w.astype(jnp.float32).reshape(NOUT, NKER, HP, HP)
    wfc = jnp.pad(wfc, ((0, NOUT_PAD - NOUT), (0, 0), (0, 0), (0, JW - HP)))
    wfc = jnp.transpose(wfc, (0, 2, 1, 3)).reshape(NOUT_PAD, KFEAT)
    wfc = wfc.astype(jnp.bfloat16)

    bias = jnp.pad(fc_b.astype(jnp.float32), (0, NOUT_PAD - NOUT))
    bias_b = jnp.broadcast_to(bias[:, None], (NOUT_PAD, block_b))

    out = pl.pallas_call(
        _body,
        out_shape=jax.ShapeDtypeStruct((NOUT_PAD, b_pad), jnp.float32),
        grid=(b_pad // block_b,),
        in_specs=[
            pl.BlockSpec((block_b, 1, H, W), lambda i: (i, 0, 0, 0)),
            pl.BlockSpec((MROWS, KCOLS), lambda i: (0, 0)),
            pl.BlockSpec((NOUT_PAD, KFEAT), lambda i: (0, 0)),
            pl.BlockSpec((NOUT_PAD, block_b), lambda i: (0, 0)),
        ],
        out_specs=pl.BlockSpec((NOUT_PAD, block_b), lambda i: (0, i)),
        scratch_shapes=[
            pltpu.VMEM((H + 2, WPAD, block_b), jnp.bfloat16),
            pltpu.VMEM((KFEAT, block_b), jnp.bfloat16),
        ],
        compiler_params=pltpu.CompilerParams(
            dimension_semantics=("parallel",)),
    )(xsq, amat, wfc, bias_b)

    return jnp.transpose(out[:NOUT, :B])


def kernel(x, conv_w, fc_w, fc_b):
    return _forward(x, conv_w, fc_w, fc_b, block_b=BLOCK_B)


# speedup vs baseline: 1.0584x; 1.0584x over previous
import functools

import numpy as np

import jax
import jax.numpy as jnp
from jax import lax
from jax.experimental import pallas as pl
from jax.experimental.pallas import tpu as pltpu

H = W = 28
WPAD = 32
NKER = 5
HP = 14
JW = 16
GROUP = NKER * JW
MROWS = 4 * GROUP
KCOLS = 4 * WPAD
KFEAT = HP * GROUP
NOUT = 10
NOUT_PAD = 16
BLOCK_B = 512


def _conv_scatter_indices():
    ms, ks, cs, hs, ws = [], [], [], [], []
    for g in range(4):
        r, par = g // 2, g % 2
        for co in range(NKER):
            for wq in range(HP):
                m = g * GROUP + co * JW + wq
                for kh in range(3):
                    for kw in range(3):
                        wi = 2 * wq + par - 1 + kw
                        if 0 <= wi < W:
                            ms.append(m)
                            ks.append((r + kh) * WPAD + wi)
                            cs.append(co)
                            hs.append(kh)
                            ws.append(kw)
    return (np.array(ms), np.array(ks), np.array(cs), np.array(hs),
            np.array(ws))

_MIDX, _KIDX, _CIDX, _HIDX, _WIDX = _conv_scatter_indices()


def _body(x_ref,
          a_ref,
          wfc_ref,
          bias_ref,
          out_ref,
          xp_ref,
          feat_ref):
    bb = x_ref.shape[0]
    row = lax.broadcasted_iota(jnp.int32, (bb, bb), 0)
    col = lax.broadcasted_iota(jnp.int32, (bb, bb), 1)
    eye = (row == col).astype(jnp.bfloat16)

    xbf = x_ref[...]
    xt = lax.dot_general(xbf, eye, (((0,), (0,)), ((), ())),
                         preferred_element_type=jnp.float32)

    xp_ref[0] = jnp.zeros((WPAD, bb), jnp.bfloat16)
    xp_ref[H + 1] = jnp.zeros((WPAD, bb), jnp.bfloat16)
    xp_ref[1:H + 1, W:, :] = jnp.zeros((H, WPAD - W, bb), jnp.bfloat16)
    xp_ref[1:H + 1, 0:W, :] = xt.astype(jnp.bfloat16)

    for i in range(HP):
        xs = xp_ref[2 * i:2 * i + 4].reshape(KCOLS, bb)
        s = jnp.dot(a_ref[...], xs,
                    preferred_element_type=jnp.float32)
        p = jnp.maximum(jnp.maximum(s[0:GROUP], s[GROUP:2 * GROUP]),
                        jnp.maximum(s[2 * GROUP:3 * GROUP], s[3 * GROUP:]))
        p = jnp.maximum(p, 0.0)
        feat_ref[i * GROUP:(i + 1) * GROUP, :] = p.astype(jnp.bfloat16)

    out_ref[...] = (
        jnp.dot(wfc_ref[...], feat_ref[...],
                preferred_element_type=jnp.float32) + bias_ref[...])


@functools.partial(jax.jit, static_argnames=("block_b",))
def _forward(x, conv_w, fc_w, fc_b, *, block_b=BLOCK_B):
    B = x.shape[0]
    assert x.shape[1:] == (1, H, W)
    b_pad = ((B + block_b - 1) // block_b) * block_b

    xsq = x[:, 0].astype(jnp.bfloat16)
    if b_pad != B:
        xsq = jnp.pad(xsq, ((0, b_pad - B), (0, 0), (0, 0)))

    cw = conv_w.astype(jnp.float32)
    vals = cw[_CIDX, 0, _HIDX, _WIDX]
    amat = jnp.zeros((MROWS, KCOLS), jnp.float32).at[_MIDX, _KIDX].set(vals)
    amat = amat.astype(jnp.bfloat16)

    wfc = fc_w.astype(jnp.float32).reshape(NOUT, NKER, HP, HP)
    wfc = jnp.pad(wfc, ((0, NOUT_PAD - NOUT), (0, 0), (0, 0), (0, JW - HP)))
    wfc = jnp.transpose(wfc, (0, 2, 1, 3)).reshape(NOUT_PAD, KFEAT)
    wfc = wfc.astype(jnp.bfloat16)

    bias = jnp.pad(fc_b.astype(jnp.float32), (0, NOUT_PAD - NOUT))
    bias_b = jnp.broadcast_to(bias[:, None], (NOUT_PAD, block_b))

    out = pl.pallas_call(
        _body,
        out_shape=jax.ShapeDtypeStruct((NOUT_PAD, b_pad), jnp.float32),
        grid=(b_pad // block_b,),
        in_specs=[
            pl.BlockSpec((block_b, H, W), lambda i: (i, 0, 0)),
            pl.BlockSpec((MROWS, KCOLS), lambda i: (0, 0)),
            pl.BlockSpec((NOUT_PAD, KFEAT), lambda i: (0, 0)),
            pl.BlockSpec((NOUT_PAD, block_b), lambda i: (0, 0)),
        ],
        out_specs=pl.BlockSpec((NOUT_PAD, block_b), lambda i: (0, i)),
        scratch_shapes=[
            pltpu.VMEM((H + 2, WPAD, block_b), jnp.bfloat16),
            pltpu.VMEM((KFEAT, block_b), jnp.bfloat16),
        ],
        compiler_params=pltpu.CompilerParams(
            dimension_semantics=("parallel",)),
    )(xsq, amat, wfc, bias_b)

    return jnp.transpose(out[:NOUT, :B])


def kernel(x, conv_w, fc_w, fc_b):
    return _forward(x, conv_w, fc_w, fc_b, block_b=BLOCK_B)


# speedup vs baseline: 1.6504x; 1.5593x over previous
import functools

import numpy as np

import jax
import jax.numpy as jnp
from jax import lax
from jax.experimental import pallas as pl
from jax.experimental.pallas import tpu as pltpu

H = W = 28
WPAD = 32
NKER = 5
HP = 14
JW = 16
GROUP = NKER * JW
MROWS = 4 * GROUP
MROWS2 = 2 * MROWS
KCOLS2 = 8 * WPAD
KFEAT = HP * GROUP
NOUT = 10
NOUT_PAD = 16
BLOCK_B = 512


def _conv_scatter_indices():
    ms, ks, cs, hs, ws = [], [], [], [], []
    for g in range(4):
        r, par = g // 2, g % 2
        for co in range(NKER):
            for wq in range(HP):
                m = g * GROUP + co * JW + wq
                for kh in range(3):
                    for kw in range(3):
                        wi = 2 * wq + par - 1 + kw
                        if 0 <= wi < W:
                            ms.append(m)
                            ks.append((r + kh) * WPAD + wi)
                            cs.append(co)
                            hs.append(kh)
                            ws.append(kw)
    return (np.array(ms), np.array(ks), np.array(cs), np.array(hs),
            np.array(ws))

_MIDX, _KIDX, _CIDX, _HIDX, _WIDX = _conv_scatter_indices()


def _body(x_ref,
          a_ref,
          wfc_ref,
          bias_ref,
          out_ref,
          xp_ref,
          feat_ref):
    bb = x_ref.shape[0]
    row = lax.broadcasted_iota(jnp.int32, (bb, bb), 0)
    col = lax.broadcasted_iota(jnp.int32, (bb, bb), 1)
    eye = (row == col).astype(jnp.bfloat16)

    xbf = x_ref[...].astype(jnp.bfloat16)
    xt = lax.dot_general(xbf, eye, (((0,), (0,)), ((), ())),
                         preferred_element_type=jnp.float32)

    xp_ref[0] = jnp.zeros((WPAD, bb), jnp.bfloat16)
    xp_ref[H + 1:] = jnp.zeros((3, WPAD, bb), jnp.bfloat16)
    xp_ref[1:H + 1, W:, :] = jnp.zeros((H, WPAD - W, bb), jnp.bfloat16)
    xp_ref[1:H + 1, 0:W, :] = xt.astype(jnp.bfloat16)

    for j in range(HP // 2):
        xs = xp_ref[4 * j:4 * j + 8].reshape(KCOLS2, bb)
        s = jnp.dot(a_ref[...], xs,
                    preferred_element_type=jnp.float32)
        for t in range(2):
            st = s[t * MROWS:(t + 1) * MROWS]
            p = jnp.maximum(
                jnp.maximum(st[0:GROUP], st[GROUP:2 * GROUP]),
                jnp.maximum(st[2 * GROUP:3 * GROUP], st[3 * GROUP:]))
            p = jnp.maximum(p, 0.0)
            feat_ref[(2 * j + t) * GROUP:(2 * j + t + 1) * GROUP, :] = (
                p.astype(jnp.bfloat16))

    out_ref[...] = (
        jnp.dot(wfc_ref[...], feat_ref[...],
                preferred_element_type=jnp.float32) + bias_ref[...])


@functools.partial(jax.jit, static_argnames=("block_b",))
def _forward(x, conv_w, fc_w, fc_b, *, block_b=BLOCK_B):
    B = x.shape[0]
    assert x.shape[1:] == (1, H, W)
    b_pad = ((B + block_b - 1) // block_b) * block_b

    xsq = x[:, 0].astype(jnp.float32)
    if b_pad != B:
        xsq = jnp.pad(xsq, ((0, b_pad - B), (0, 0), (0, 0)))

    cw = conv_w.astype(jnp.float32)
    vals = cw[_CIDX, 0, _HIDX, _WIDX]
    vals2 = jnp.concatenate([vals, vals])
    m2 = np.concatenate([_MIDX, _MIDX + MROWS])
    k2 = np.concatenate([_KIDX, _KIDX + 2 * WPAD])
    amat = jnp.zeros((MROWS2, KCOLS2), jnp.float32).at[m2, k2].set(vals2)
    amat = amat.astype(jnp.bfloat16)

    wfc = fc_w.astype(jnp.float32).reshape(NOUT, NKER, HP, HP)
    wfc = jnp.pad(wfc, ((0, NOUT_PAD - NOUT), (0, 0), (0, 0), (0, JW - HP)))
    wfc = jnp.transpose(wfc, (0, 2, 1, 3)).reshape(NOUT_PAD, KFEAT)
    wfc = wfc.astype(jnp.bfloat16)

    bias = jnp.pad(fc_b.astype(jnp.float32), (0, NOUT_PAD - NOUT))
    bias_b = jnp.broadcast_to(bias[:, None], (NOUT_PAD, block_b))

    out = pl.pallas_call(
        _body,
        out_shape=jax.ShapeDtypeStruct((NOUT_PAD, b_pad), jnp.float32),
        grid=(b_pad // block_b,),
        in_specs=[
            pl.BlockSpec((block_b, H, W), lambda i: (i, 0, 0)),
            pl.BlockSpec((MROWS2, KCOLS2), lambda i: (0, 0)),
            pl.BlockSpec((NOUT_PAD, KFEAT), lambda i: (0, 0)),
            pl.BlockSpec((NOUT_PAD, block_b), lambda i: (0, 0)),
        ],
        out_specs=pl.BlockSpec((NOUT_PAD, block_b), lambda i: (0, i)),
        scratch_shapes=[
            pltpu.VMEM((H + 4, WPAD, block_b), jnp.bfloat16),
            pltpu.VMEM((KFEAT, block_b), jnp.bfloat16),
        ],
        compiler_params=pltpu.CompilerParams(
            dimension_semantics=("parallel",)),
    )(xsq, amat, wfc, bias_b)

    return jnp.transpose(out[:NOUT, :B])


def kernel(x, conv_w, fc_w, fc_b):
    return _forward(x, conv_w, fc_w, fc_b, block_b=BLOCK_B)


# speedup vs baseline: 1.8335x; 1.1109x over previous
import functools

import numpy as np

import jax
import jax.numpy as jnp
from jax import lax
from jax.experimental import pallas as pl
from jax.experimental.pallas import tpu as pltpu

H = W = 28
WPAD = 32
NKER = 5
HP = 14
JW = 16
GROUP = NKER * JW
MROWS = 4 * GROUP
KCOLS = 4 * WPAD
KFEAT = HP * GROUP
NOUT = 10
NOUT_PAD = 16
BLOCK_B = 512


def _conv_scatter_indices():
    ms, ks, cs, hs, ws = [], [], [], [], []
    for g in range(4):
        r, par = g // 2, g % 2
        for co in range(NKER):
            for wq in range(HP):
                m = g * GROUP + co * JW + wq
                for kh in range(3):
                    for kw in range(3):
                        wi = 2 * wq + par - 1 + kw
                        if 0 <= wi < W:
                            ms.append(m)
                            ks.append((r + kh) * WPAD + wi)
                            cs.append(co)
                            hs.append(kh)
                            ws.append(kw)
    return (np.array(ms), np.array(ks), np.array(cs), np.array(hs),
            np.array(ws))

_MIDX, _KIDX, _CIDX, _HIDX, _WIDX = _conv_scatter_indices()


def _body(x_ref,
          a_ref,
          wfc_ref,
          bias_ref,
          out_ref,
          xp_ref,
          feat_ref):
    bb = x_ref.shape[0]
    row = lax.broadcasted_iota(jnp.int32, (bb, bb), 0)
    col = lax.broadcasted_iota(jnp.int32, (bb, bb), 1)
    eye = (row == col).astype(jnp.bfloat16)

    xbf = x_ref[...].astype(jnp.bfloat16)
    xt = lax.dot_general(xbf, eye, (((0,), (0,)), ((), ())),
                         preferred_element_type=jnp.float32)

    xp_ref[0] = jnp.zeros((WPAD, bb), jnp.bfloat16)
    xp_ref[H + 1] = jnp.zeros((WPAD, bb), jnp.bfloat16)
    xp_ref[1:H + 1, W:, :] = jnp.zeros((H, WPAD - W, bb), jnp.bfloat16)
    xp_ref[1:H + 1, 0:W, :] = xt.astype(jnp.bfloat16)

    for i in range(HP):
        xs = xp_ref[2 * i:2 * i + 4].reshape(KCOLS, bb)
        s = jnp.dot(a_ref[...], xs,
                    preferred_element_type=jnp.float32)
        p = jnp.maximum(jnp.maximum(s[0:GROUP], s[GROUP:2 * GROUP]),
                        jnp.maximum(s[2 * GROUP:3 * GROUP], s[3 * GROUP:]))
        p = jnp.maximum(p, 0.0)
        feat_ref[i * GROUP:(i + 1) * GROUP, :] = p.astype(jnp.bfloat16)

    out_ref[...] = (
        jnp.dot(wfc_ref[...], feat_ref[...],
                preferred_element_type=jnp.float32) + bias_ref[...])


@functools.partial(jax.jit, static_argnames=("block_b",))
def _forward(x, conv_w, fc_w, fc_b, *, block_b=BLOCK_B):
    B = x.shape[0]
    assert x.shape[1:] == (1, H, W)
    b_pad = ((B + block_b - 1) // block_b) * block_b

    xsq = x[:, 0].astype(jnp.float32)
    if b_pad != B:
        xsq = jnp.pad(xsq, ((0, b_pad - B), (0, 0), (0, 0)))

    cw = conv_w.astype(jnp.float32)
    vals = cw[_CIDX, 0, _HIDX, _WIDX]
    amat = jnp.zeros((MROWS, KCOLS), jnp.float32).at[_MIDX, _KIDX].set(vals)
    amat = amat.astype(jnp.bfloat16)

    wfc = fc_w.astype(jnp.float32).reshape(NOUT, NKER, HP, HP)
    wfc = jnp.pad(wfc, ((0, NOUT_PAD - NOUT), (0, 0), (0, 0), (0, JW - HP)))
    wfc = jnp.transpose(wfc, (0, 2, 1, 3)).reshape(NOUT_PAD, KFEAT)
    wfc = wfc.astype(jnp.bfloat16)

    bias = jnp.pad(fc_b.astype(jnp.float32), (0, NOUT_PAD - NOUT))
    bias_b = jnp.broadcast_to(bias[:, None], (NOUT_PAD, block_b))

    out = pl.pallas_call(
        _body,
        out_shape=jax.ShapeDtypeStruct((NOUT_PAD, b_pad), jnp.float32),
        grid=(b_pad // block_b,),
        in_specs=[
            pl.BlockSpec((block_b, H, W), lambda i: (i, 0, 0)),
            pl.BlockSpec((MROWS, KCOLS), lambda i: (0, 0)),
            pl.BlockSpec((NOUT_PAD, KFEAT), lambda i: (0, 0)),
            pl.BlockSpec((NOUT_PAD, block_b), lambda i: (0, 0)),
        ],
        out_specs=pl.BlockSpec((NOUT_PAD, block_b), lambda i: (0, i)),
        scratch_shapes=[
            pltpu.VMEM((H + 2, WPAD, block_b), jnp.bfloat16),
            pltpu.VMEM((KFEAT, block_b), jnp.bfloat16),
        ],
        compiler_params=pltpu.CompilerParams(
            dimension_semantics=("parallel",)),
    )(xsq, amat, wfc, bias_b)

    return jnp.transpose(out[:NOUT, :B])


def kernel(x, conv_w, fc_w, fc_b):
    return _forward(x, conv_w, fc_w, fc_b, block_b=BLOCK_B)
